# baseline (speedup 1.0000x reference)
import functools

import jax
import jax.numpy as jnp
from jax import lax
from jax.experimental import pallas as pl
from jax.experimental.pallas import tpu as pltpu
from jax.experimental.pallas import tpu_sc as plsc

N = 10000
E = 320000
D = 128
H = 64
R = 8

RP = 5
NC = 2
NS = 16
NW = NC * NS
C = 80
NCH = 129
EPT = NCH * C
EPAD = NW * EPT - E - N
NBUF = 4
ROWS_PT = 640
NP = NS * ROWS_PT


def _i0():
    return jnp.int32(0)


def _z_body(x_ref, w_ref, b_ref, z_ref):
    za = lax.dot_general(
        x_ref[...], w_ref[0, 0], (((1,), (1,)), ((), ())),
        preferred_element_type=jnp.float32,
    ) + b_ref[0, 0]
    zb = lax.dot_general(
        x_ref[...], w_ref[0, 1], (((1,), (1,)), ((), ())),
        preferred_element_type=jnp.float32,
    ) + b_ref[0, 1]
    z_ref[...] = jnp.concatenate([za, zb], axis=1)


def _z_packed(x, w_pairs, b_pairs):
    return pl.pallas_call(
        _z_body,
        grid=(RP,),
        in_specs=[
            pl.BlockSpec((N, D), lambda r: (_i0(), _i0())),
            pl.BlockSpec((1, 2, H, D), lambda r: (r, _i0(), _i0(), _i0())),
            pl.BlockSpec((1, 2, 1, H), lambda r: (r, _i0(), _i0(), _i0())),
        ],
        out_specs=pl.BlockSpec((N, 2 * H), lambda r: (r, _i0())),
        out_shape=jax.ShapeDtypeStruct((RP * N, 2 * H), jnp.float32),
    )(x, w_pairs, b_pairs)


def _sc_body(z_hbm, row_hbm, col_hbm, typ_hbm, zer_hbm, out_hbm,
             gidx_v, col_v, row_v, typ_v, rows_v, acc_sh, *sems):
    c = lax.axis_index("c")
    s = lax.axis_index("s")
    wid = c * NS + s

    pltpu.sync_copy(row_hbm.at[wid], row_v)
    pltpu.sync_copy(col_hbm.at[wid], col_v)
    pltpu.sync_copy(typ_hbm.at[wid], typ_v)

    pltpu.sync_copy(zer_hbm.at[pl.ds(s * ROWS_PT, ROWS_PT)],
                    acc_sh.at[pl.ds(s * ROWS_PT, ROWS_PT)])

    def _gi(j, carry):
        for k in range(C // 16):
            sl = pl.ds(k * 16, 16)
            tv = typ_v[j, sl]
            gidx_v[j, sl] = ((tv >> 1) * (2 * N) + (tv & 1)
                             + row_v[j, sl] * 2)
        return carry

    lax.fori_loop(jnp.int32(0), jnp.int32(NCH), _gi, jnp.int32(0))
    plsc.subcore_barrier()

    for b in range(NBUF):
        jb = jnp.int32(b)
        pltpu.async_copy(z_hbm.at[gidx_v.at[jb]], rows_v.at[jb], sems[b])

    def _group(kk, carry):
        g0 = kk * NBUF
        for b in range(NBUF):
            j = g0 + b
            jb = jnp.int32(b)

            @pl.when(j < NCH)
            def _():
                pltpu.make_async_copy(
                    z_hbm.at[gidx_v.at[jb]], rows_v.at[jb], sems[b]).wait()
                pltpu.sync_copy(rows_v.at[jb], acc_sh.at[col_v.at[j]], add=True)

                @pl.when(j + NBUF < NCH)
                def _():
                    pltpu.async_copy(
                        z_hbm.at[gidx_v.at[j + NBUF]], rows_v.at[jb], sems[b])
        return carry

    ngroups = (NCH + NBUF - 1) // NBUF
    lax.fori_loop(jnp.int32(0), jnp.int32(ngroups), _group, jnp.int32(0))
    plsc.subcore_barrier()

    pltpu.sync_copy(acc_sh.at[pl.ds(s * ROWS_PT, ROWS_PT)],
                    out_hbm.at[c, pl.ds(s * ROWS_PT, ROWS_PT)])


_scatter = functools.partial(
    pl.kernel,
    out_type=jax.ShapeDtypeStruct((NC, NP, H), jnp.float32),
    mesh=plsc.VectorSubcoreMesh(core_axis_name="c", subcore_axis_name="s"),
    compiler_params=pltpu.CompilerParams(use_tc_tiling_on_sc=False),
    scratch_types=[
        pltpu.VMEM((NCH, C), jnp.int32),
        pltpu.VMEM((NCH, C), jnp.int32),
        pltpu.VMEM((NCH, C), jnp.int32),
        pltpu.VMEM((NCH, C), jnp.int32),
        pltpu.VMEM((NBUF, C, H), jnp.float32),
        pltpu.VMEM_SHARED((NP, H), jnp.float32),
    ] + [pltpu.SemaphoreType.DMA] * NBUF,
)(_sc_body)


def _comb_body(a_ref, w_ref, b_ref, o_ref):
    h = jnp.maximum(a_ref[0] + a_ref[1], 0.0)
    o_ref[...] = (
        lax.dot_general(
            h, w_ref[...], (((1,), (0,)), ((), ())),
            preferred_element_type=jnp.float32,
        )
        + b_ref[0, 0]
    )


def _combine(acc_p, w2, out_b):
    return pl.pallas_call(
        _comb_body,
        grid=(1,),
        in_specs=[
            pl.BlockSpec((NC, NP // 2, 2 * H), lambda g: (_i0(), _i0(), _i0())),
            pl.BlockSpec((2 * H, 2), lambda g: (_i0(), _i0())),
            pl.BlockSpec((1, 1), lambda g: (_i0(), _i0())),
        ],
        out_specs=pl.BlockSpec((NP // 2, 2), lambda g: (_i0(), _i0())),
        out_shape=jax.ShapeDtypeStruct((NP // 2, 2), jnp.float32),
    )(acc_p, w2, out_b.reshape(1, 1))


def kernel(x, edge_index, edge_type, rel_W, rel_b, self_W, self_b, out_W, out_b):
    x = x.astype(jnp.float32)
    selfv = jnp.arange(N, dtype=jnp.int32)
    row = jnp.concatenate([edge_index[0].astype(jnp.int32), selfv,
                           jnp.zeros((EPAD,), jnp.int32)])
    col = jnp.concatenate([edge_index[1].astype(jnp.int32), selfv,
                           jnp.full((EPAD,), N, jnp.int32)])
    typ = jnp.concatenate([edge_type.astype(jnp.int32),
                           jnp.full((N,), R, jnp.int32),
                           jnp.zeros((EPAD,), jnp.int32)])
    row = row.reshape(NW, NCH, C)
    col = col.reshape(NW, NCH, C)
    typ = typ.reshape(NW, NCH, C)

    w_pairs = jnp.concatenate(
        [rel_W, self_W[None], jnp.zeros((1, H, D), jnp.float32)]
    ).reshape(RP, 2, H, D)
    b_pairs = jnp.concatenate(
        [rel_b, self_b[None], jnp.zeros((1, H), jnp.float32)]
    ).reshape(RP, 2, 1, H)

    z_packed = _z_packed(x, w_pairs, b_pairs)
    z2d = z_packed.reshape(2 * RP * N, H)
    zeros = jnp.zeros((NP, H), jnp.float32)
    acc = _scatter(z2d, row, col, typ, zeros)
    acc_p = acc.reshape(NC, NP // 2, 2 * H)
    w2 = jnp.zeros((2 * H, 2), jnp.float32)
    w2 = w2.at[:H, 0].set(out_W[0]).at[H:, 1].set(out_W[0])
    y = _combine(acc_p, w2, out_b)
    return y.reshape(NP)[:N]


# speedup vs baseline: 1.1581x; 1.1581x over previous
import functools

import jax
import jax.numpy as jnp
from jax import lax
from jax.experimental import pallas as pl
from jax.experimental.pallas import tpu as pltpu
from jax.experimental.pallas import tpu_sc as plsc

N = 10000
E = 320000
D = 128
H = 64
R = 8

RP = 5
NC = 2
NS = 16
NW = NC * NS
C = 80
NCH = 129
EPT = NCH * C
EPAD = NW * EPT - E - N
NBUF = 4
ROWS_PT = 640
NP = NS * ROWS_PT


def _i0():
    return jnp.int32(0)


def _z_body(x_ref, w_ref, b_ref, z_ref):
    za = lax.dot_general(
        x_ref[...], w_ref[0, 0], (((1,), (1,)), ((), ())),
        preferred_element_type=jnp.float32,
    ) + b_ref[0, 0]
    zb = lax.dot_general(
        x_ref[...], w_ref[0, 1], (((1,), (1,)), ((), ())),
        preferred_element_type=jnp.float32,
    ) + b_ref[0, 1]
    z_ref[...] = jnp.concatenate([za, zb], axis=1)


def _z_packed(x, w_pairs, b_pairs):
    return pl.pallas_call(
        _z_body,
        grid=(RP,),
        in_specs=[
            pl.BlockSpec((N, D), lambda r: (_i0(), _i0())),
            pl.BlockSpec((1, 2, H, D), lambda r: (r, _i0(), _i0(), _i0())),
            pl.BlockSpec((1, 2, 1, H), lambda r: (r, _i0(), _i0(), _i0())),
        ],
        out_specs=pl.BlockSpec((N, 2 * H), lambda r: (r, _i0())),
        out_shape=jax.ShapeDtypeStruct((RP * N, 2 * H), jnp.float32),
    )(x, w_pairs, b_pairs)


def _sc_body(z_hbm, rct_hbm, zer_hbm, out_hbm,
             gidx_v, col_v, row_v, typ_v, rows_v, acc_sh, *sems):
    c = lax.axis_index("c")
    s = lax.axis_index("s")
    wid = c * NS + s

    pltpu.sync_copy(rct_hbm.at[jnp.int32(0), wid], row_v)
    pltpu.sync_copy(rct_hbm.at[jnp.int32(1), wid], col_v)
    pltpu.sync_copy(rct_hbm.at[jnp.int32(2), wid], typ_v)

    pltpu.sync_copy(zer_hbm.at[pl.ds(s * ROWS_PT, ROWS_PT)],
                    acc_sh.at[pl.ds(s * ROWS_PT, ROWS_PT)])

    def _gi(j, carry):
        for k in range(C // 16):
            sl = pl.ds(k * 16, 16)
            tv = typ_v[j, sl]
            gidx_v[j, sl] = ((tv >> 1) * (2 * N) + (tv & 1)
                             + row_v[j, sl] * 2)
        return carry

    lax.fori_loop(jnp.int32(0), jnp.int32(NCH), _gi, jnp.int32(0))
    plsc.subcore_barrier()

    for b in range(NBUF):
        jb = jnp.int32(b)
        pltpu.async_copy(z_hbm.at[gidx_v.at[jb]], rows_v.at[jb], sems[b])

    def _group(kk, carry):
        g0 = kk * NBUF
        for b in range(NBUF):
            j = g0 + b
            jb = jnp.int32(b)

            @pl.when(j < NCH)
            def _():
                pltpu.make_async_copy(
                    z_hbm.at[gidx_v.at[jb]], rows_v.at[jb], sems[b]).wait()
                pltpu.sync_copy(rows_v.at[jb], acc_sh.at[col_v.at[j]], add=True)

                @pl.when(j + NBUF < NCH)
                def _():
                    pltpu.async_copy(
                        z_hbm.at[gidx_v.at[j + NBUF]], rows_v.at[jb], sems[b])
        return carry

    ngroups = (NCH + NBUF - 1) // NBUF
    lax.fori_loop(jnp.int32(0), jnp.int32(ngroups), _group, jnp.int32(0))
    plsc.subcore_barrier()

    pltpu.sync_copy(acc_sh.at[pl.ds(s * ROWS_PT, ROWS_PT)],
                    out_hbm.at[c, pl.ds(s * ROWS_PT, ROWS_PT)])


_scatter = functools.partial(
    pl.kernel,
    out_type=jax.ShapeDtypeStruct((NC, NP, H), jnp.float32),
    mesh=plsc.VectorSubcoreMesh(core_axis_name="c", subcore_axis_name="s"),
    compiler_params=pltpu.CompilerParams(use_tc_tiling_on_sc=False),
    scratch_types=[
        pltpu.VMEM((NCH, C), jnp.int32),
        pltpu.VMEM((NCH, C), jnp.int32),
        pltpu.VMEM((NCH, C), jnp.int32),
        pltpu.VMEM((NCH, C), jnp.int32),
        pltpu.VMEM((NBUF, C, H), jnp.float32),
        pltpu.VMEM_SHARED((NP, H), jnp.float32),
    ] + [pltpu.SemaphoreType.DMA] * NBUF,
)(_sc_body)


def _comb_body(a_ref, w_ref, b_ref, o_ref):
    h = jnp.maximum(a_ref[0] + a_ref[1], 0.0)
    o_ref[...] = (
        lax.dot_general(
            h, w_ref[...], (((1,), (0,)), ((), ())),
            preferred_element_type=jnp.float32,
        )
        + b_ref[0, 0]
    )


def _combine(acc_p, w2, out_b):
    return pl.pallas_call(
        _comb_body,
        grid=(1,),
        in_specs=[
            pl.BlockSpec((NC, NP // 2, 2 * H), lambda g: (_i0(), _i0(), _i0())),
            pl.BlockSpec((2 * H, 2), lambda g: (_i0(), _i0())),
            pl.BlockSpec((1, 1), lambda g: (_i0(), _i0())),
        ],
        out_specs=pl.BlockSpec((NP // 2, 2), lambda g: (_i0(), _i0())),
        out_shape=jax.ShapeDtypeStruct((NP // 2, 2), jnp.float32),
    )(acc_p, w2, out_b.reshape(1, 1))


def kernel(x, edge_index, edge_type, rel_W, rel_b, self_W, self_b, out_W, out_b):
    x = x.astype(jnp.float32)
    selfv = jnp.arange(N, dtype=jnp.int32)
    main = jnp.stack([edge_index[0], edge_index[1], edge_type]).astype(jnp.int32)
    selfb = jnp.stack([selfv, selfv, jnp.full((N,), R, jnp.int32)])
    padb = jnp.stack([jnp.zeros((EPAD,), jnp.int32),
                      jnp.full((EPAD,), N, jnp.int32),
                      jnp.zeros((EPAD,), jnp.int32)])
    rct = jnp.concatenate([main, selfb, padb], axis=1).reshape(3, NW, NCH, C)

    w_pairs = jnp.concatenate(
        [rel_W, self_W[None], jnp.zeros((1, H, D), jnp.float32)]
    ).reshape(RP, 2, H, D)
    b_pairs = jnp.concatenate(
        [rel_b, self_b[None], jnp.zeros((1, H), jnp.float32)]
    ).reshape(RP, 2, 1, H)

    z_packed = _z_packed(x, w_pairs, b_pairs)
    z2d = z_packed.reshape(2 * RP * N, H)
    zeros = jnp.zeros((NP, H), jnp.float32)
    acc = _scatter(z2d, rct, zeros)
    acc_p = acc.reshape(NC, NP // 2, 2 * H)
    w2 = jnp.zeros((2 * H, 2), jnp.float32)
    w2 = w2.at[:H, 0].set(out_W[0]).at[H:, 1].set(out_W[0])
    y = _combine(acc_p, w2, out_b)
    return y.reshape(NP)[:N]
